# baseline (speedup 1.0000x reference)
import functools
import jax
import jax.numpy as jnp
from jax import lax
from jax.experimental import pallas as pl
from jax.experimental.pallas import tpu as pltpu
from jax.experimental.pallas import tpu_sc as plsc

D_MODEL = 64
LANES = 128
NUM_CORES = 2
NUM_SUBCORES = 16
NUM_WORKERS = NUM_CORES * NUM_SUBCORES
UNITS_PER_WORKER = 200


def _unit_math(u):
    s = lax.shift_right_logical(u, 5)
    tb = lax.bitwise_and(u, 31)
    return s, tb


def _body(xt_hbm, te_hbm, out_hbm, idx_v, rows_v, tile_v, gsem, wsem, isem):
    wid = lax.axis_index("s") * NUM_CORES + lax.axis_index("c")
    u0 = wid * UNITS_PER_WORKER

    base_iota = lax.iota(jnp.int32, 16)
    perm = [lax.bitwise_and(base_iota + c, 15) for c in range(16)]

    def _idx_src(g):
        s, tb = _unit_math(u0 + g)
        return xt_hbm.at[lax.shift_right_logical(s, 3), tb,
                         lax.bitwise_and(s, 7)]

    def start_idx(g, b):
        pltpu.async_copy(_idx_src(g), idx_v.at[b], isem.at[b])

    def wait_idx(g, b):
        pltpu.make_async_copy(_idx_src(g), idx_v.at[b], isem.at[b]).wait()

    def start_gather(b):
        pltpu.async_copy(te_hbm.at[idx_v.at[b]], rows_v.at[b], gsem.at[b])

    def wait_gather(b):
        pltpu.make_async_copy(te_hbm.at[idx_v.at[b]], rows_v.at[b],
                              gsem.at[b]).wait()

    def transpose(b):
        A = rows_v.at[b]
        B = tile_v.at[b]

        @plsc.parallel_loop(0, 32, unroll=4)
        def _(blk):
            row0 = lax.shift_right_logical(blk, 2) * 16
            col0 = lax.bitwise_and(blk, 3) * 16
            rvec = base_iota + row0
            for c in range(16):
                cvec = perm[c] + col0
                v = plsc.load_gather(A, [rvec, cvec])
                plsc.store_scatter(B, [cvec, rvec], v)

    def start_wo(g, b):
        s, tb = _unit_math(u0 + g)
        for td in range(8):
            pltpu.async_copy(tile_v.at[b, pl.ds(td * 8, 8)],
                             out_hbm.at[s, td, tb], wsem.at[b])

    def wait_wo(g, b):
        s, tb = _unit_math(u0 + g)
        for td in range(8):
            pltpu.make_async_copy(tile_v.at[b, pl.ds(td * 8, 8)],
                                  out_hbm.at[s, td, tb], wsem.at[b]).wait()

    start_idx(0, 0)
    start_idx(1, 1)
    wait_idx(0, 0)
    start_gather(0)

    def unit(g, b, first=False, no_idx=False, no_gather=False):
        wait_gather(b)
        if not no_idx:
            start_idx(g + 2, b)
        if not no_gather:
            wait_idx(g + 1, b ^ 1)
            start_gather(b ^ 1)
        if not first:
            wait_wo(g - 2, b)
        transpose(b)
        start_wo(g, b)

    unit(0, 0, first=True)
    unit(1, 1, first=True)

    def outer(go, carry):
        g = go * 2
        unit(g, 0)
        unit(g + 1, 1)
        return carry

    lax.fori_loop(1, UNITS_PER_WORKER // 2 - 1, outer, 0, unroll=False)

    n = UNITS_PER_WORKER
    unit(n - 2, 0, no_idx=True)
    unit(n - 1, 1, no_idx=True, no_gather=True)
    wait_wo(n - 2, 0)
    wait_wo(n - 1, 1)


def kernel(x, te):
    batch, seq = x.shape
    assert batch % LANES == 0 and D_MODEL == te.shape[1]
    n_btiles = batch // LANES

    x4d = (x.astype(jnp.int32).T.reshape(seq // 8, 8, n_btiles, LANES)
           .transpose((0, 2, 1, 3)))

    mesh = plsc.VectorSubcoreMesh(core_axis_name="c", subcore_axis_name="s")
    run = pl.kernel(
        _body,
        out_type=jax.ShapeDtypeStruct((seq, 8, n_btiles, 8, LANES),
                                      jnp.float32),
        mesh=mesh,
        scratch_types=[
            pltpu.VMEM((2, LANES), jnp.int32),
            pltpu.VMEM((2, LANES, D_MODEL), jnp.float32),
            pltpu.VMEM((2, D_MODEL, LANES), jnp.float32),
            pltpu.SemaphoreType.DMA((2,)),
            pltpu.SemaphoreType.DMA((2,)),
            pltpu.SemaphoreType.DMA((2,)),
        ],
        compiler_params=pltpu.CompilerParams(use_tc_tiling_on_sc=False,
                                             needs_layout_passes=False),
    )
    out5 = run(x4d, te)
    return out5.transpose((2, 4, 0, 1, 3)).reshape(batch, seq, D_MODEL)


# speedup vs baseline: 1.0198x; 1.0198x over previous
import functools
import jax
import jax.numpy as jnp
from jax import lax
from jax.experimental import pallas as pl
from jax.experimental.pallas import tpu as pltpu
from jax.experimental.pallas import tpu_sc as plsc

D_MODEL = 64
LANES = 128
NUM_CORES = 2
NUM_SUBCORES = 16
NUM_WORKERS = NUM_CORES * NUM_SUBCORES
UNITS_PER_WORKER = 200


def _unit_math(u):
    s = lax.shift_right_logical(u, 5)
    tb = lax.bitwise_and(u, 31)
    return s, tb


def _body(xt_hbm, te_hbm, out_hbm, idx_v, rows_v, tile_v, gsem, wsem, isem):
    wid = lax.axis_index("s") * NUM_CORES + lax.axis_index("c")
    u0 = wid * UNITS_PER_WORKER

    base_iota = lax.iota(jnp.int32, 16)
    perm = [lax.bitwise_and(base_iota + c, 15) for c in range(16)]

    def _idx_src(g):
        s, tb = _unit_math(u0 + g)
        return xt_hbm.at[lax.shift_right_logical(s, 3), tb,
                         lax.bitwise_and(s, 7)]

    def start_idx(g, b):
        pltpu.async_copy(_idx_src(g), idx_v.at[b], isem.at[b])

    def wait_idx(g, b):
        pltpu.make_async_copy(_idx_src(g), idx_v.at[b], isem.at[b]).wait()

    def start_gather(b):
        pltpu.async_copy(te_hbm.at[idx_v.at[b]], rows_v.at[b], gsem.at[b])

    def wait_gather(b):
        pltpu.make_async_copy(te_hbm.at[idx_v.at[b]], rows_v.at[b],
                              gsem.at[b]).wait()

    def transpose(b):
        A = rows_v.at[b]
        B = tile_v.at[b]

        @plsc.parallel_loop(0, 32, unroll=2)
        def _(blk):
            row0 = lax.shift_right_logical(blk, 2) * 16
            col0 = lax.bitwise_and(blk, 3) * 16
            rvec = base_iota + row0
            for c in range(16):
                cvec = perm[c] + col0
                v = plsc.load_gather(A, [rvec, cvec])
                plsc.store_scatter(B, [cvec, rvec], v)

    def start_wo(g, b):
        s, tb = _unit_math(u0 + g)
        for td in range(8):
            pltpu.async_copy(tile_v.at[b, pl.ds(td * 8, 8)],
                             out_hbm.at[s, td, tb], wsem.at[b])

    def wait_wo(g, b):
        s, tb = _unit_math(u0 + g)
        for td in range(8):
            pltpu.make_async_copy(tile_v.at[b, pl.ds(td * 8, 8)],
                                  out_hbm.at[s, td, tb], wsem.at[b]).wait()

    start_idx(0, 0)
    start_idx(1, 1)
    wait_idx(0, 0)
    start_gather(0)

    def unit(g, b, first=False, no_idx=False, no_gather=False):
        wait_gather(b)
        if not no_idx:
            start_idx(g + 2, b)
        if not no_gather:
            wait_idx(g + 1, b ^ 1)
            start_gather(b ^ 1)
        if not first:
            wait_wo(g - 2, b)
        transpose(b)
        start_wo(g, b)

    unit(0, 0, first=True)
    unit(1, 1, first=True)

    def outer(go, carry):
        g = go * 2
        unit(g, 0)
        unit(g + 1, 1)
        return carry

    lax.fori_loop(1, UNITS_PER_WORKER // 2 - 1, outer, 0, unroll=False)

    n = UNITS_PER_WORKER
    unit(n - 2, 0, no_idx=True)
    unit(n - 1, 1, no_idx=True, no_gather=True)
    wait_wo(n - 2, 0)
    wait_wo(n - 1, 1)


def kernel(x, te):
    batch, seq = x.shape
    assert batch % LANES == 0 and D_MODEL == te.shape[1]
    n_btiles = batch // LANES

    x4d = (x.astype(jnp.int32).T.reshape(seq // 8, 8, n_btiles, LANES)
           .transpose((0, 2, 1, 3)))

    mesh = plsc.VectorSubcoreMesh(core_axis_name="c", subcore_axis_name="s")
    run = pl.kernel(
        _body,
        out_type=jax.ShapeDtypeStruct((seq, 8, n_btiles, 8, LANES),
                                      jnp.float32),
        mesh=mesh,
        scratch_types=[
            pltpu.VMEM((2, LANES), jnp.int32),
            pltpu.VMEM((2, LANES, D_MODEL), jnp.float32),
            pltpu.VMEM((2, D_MODEL, LANES), jnp.float32),
            pltpu.SemaphoreType.DMA((2,)),
            pltpu.SemaphoreType.DMA((2,)),
            pltpu.SemaphoreType.DMA((2,)),
        ],
        compiler_params=pltpu.CompilerParams(use_tc_tiling_on_sc=False,
                                             needs_layout_passes=False),
    )
    out5 = run(x4d, te)
    return out5.transpose((2, 4, 0, 1, 3)).reshape(batch, seq, D_MODEL)
